# baseline (speedup 1.0000x reference)
import jax
import jax.numpy as jnp
from jax import lax
from jax.experimental import pallas as pl
from jax.experimental.pallas import tpu as pltpu
from jax.experimental.pallas import tpu_sc as plsc

_TILE_N = 1024
_TILE_W = 2048
_GATHER_W = 256


def _select_kernel(xyz_t_ref, qneg2_ref, sxyz_ref, sneg_ref, idx_ref):
    b = pl.program_id(0)
    q = xyz_t_ref[0]
    s = sxyz_ref[0]
    T = q.shape[0]
    S = s.shape[1]

    qc = [q[:, c:c + 1] for c in range(3)]
    sc = [s[c:c + 1, :] for c in range(3)]
    qq = qc[0] * qc[0] + qc[1] * qc[1] + qc[2] * qc[2]
    ss = sc[0] * sc[0] + sc[1] * sc[1] + sc[2] * sc[2]

    crossm = lax.dot_general(
        qneg2_ref[0], sneg_ref[0], (((1,), (0,)), ((), ())),
        preferred_element_type=jnp.float32)
    d_sel = (qq + ss) + crossm

    iota_f = lax.broadcasted_iota(jnp.int32, (T, S), 1).astype(jnp.float32)
    inf = jnp.float32(jnp.inf)
    big = jnp.float32(1e9)
    base_id = b * S

    ids = []
    for k in range(3):
        mk = jnp.min(d_sel, axis=1, keepdims=True)
        cand = jnp.where(d_sel == mk, iota_f, big)
        ik = jnp.min(cand, axis=1, keepdims=True)
        ids.append(ik.astype(jnp.int32) + base_id)
        if k < 2:
            d_sel = jnp.where(cand == ik, inf, d_sel)
    idx_ref[0] = jnp.concatenate(ids, axis=1)


def _combine_kernel(g_ref, xyz_ref, out_ref):
    q = xyz_ref[0]
    recip = []
    flows = []
    for k in range(3):
        arr = g_ref[0, k]
        diff = arr[0:3, :] - q
        sq = diff * diff
        dist2 = sq[0:1, :] + sq[1:2, :] + sq[2:3, :]
        recip.append(1.0 / jnp.maximum(jnp.sqrt(dist2), 1e-10))
        flows.append(arr[3:6, :])
    norm = recip[0] + recip[1] + recip[2]
    acc = (recip[0] / norm) * flows[0]
    acc = acc + (recip[1] / norm) * flows[1]
    acc = acc + (recip[2] / norm) * flows[2]
    out_ref[0] = acc


def _sc_gather(table, indices):
    M = indices.shape[0]
    mesh = plsc.VectorSubcoreMesh(core_axis_name="c", subcore_axis_name="s")
    idx2d = indices.reshape(1, M)

    @pl.kernel(out_type=jax.ShapeDtypeStruct((M, 128), table.dtype), mesh=mesh)
    def gather_kernel(x_hbm, i_hbm, o_hbm):
        def body(i_vmem, o_vmem):
            pltpu.sync_copy(x_hbm.at[i_vmem.at[0]], o_vmem)

        pltpu.emit_pipeline(
            body,
            grid=(M // _GATHER_W,),
            in_specs=[pl.BlockSpec((1, _GATHER_W),
                                   index_map=lambda i: (0, i))],
            out_specs=[pl.BlockSpec((_GATHER_W, 128),
                                    index_map=lambda i: (i, 0))],
            core_axis_name=("c", "s"),
            dimension_semantics=(pltpu.PARALLEL,),
        )(i_hbm, o_hbm)

    return gather_kernel(table, idx2d)


def kernel(xyz, sparse_xyz, sparse_flow):
    B, C, N = xyz.shape
    _, _, S = sparse_xyz.shape
    xyz_t = jnp.transpose(xyz, (0, 2, 1))
    qb = xyz_t.astype(jnp.bfloat16) * jnp.bfloat16(-2.0)
    qneg2 = jnp.pad(qb, ((0, 0), (0, 0), (0, 5)))
    sneg = jnp.pad(sparse_xyz.astype(jnp.bfloat16),
                   ((0, 0), (0, 5), (0, 0)))

    table = jnp.concatenate(
        [jnp.transpose(sparse_xyz, (0, 2, 1)),
         jnp.transpose(sparse_flow, (0, 2, 1))], axis=2)
    table = jnp.pad(table, ((0, 0), (0, 0), (0, 122)))
    table = table.reshape(B * S, 128)

    def run_half(xyz_t_h, qneg2_h, xyz_h):
        H = xyz_t_h.shape[1]
        idx = pl.pallas_call(
            _select_kernel,
            grid=(B, H // _TILE_N),
            in_specs=[
                pl.BlockSpec((1, _TILE_N, C), lambda b, i: (b, i, 0)),
                pl.BlockSpec((1, _TILE_N, 8), lambda b, i: (b, i, 0)),
                pl.BlockSpec((1, C, S), lambda b, i: (b, 0, 0)),
                pl.BlockSpec((1, 8, S), lambda b, i: (b, 0, 0)),
            ],
            out_specs=pl.BlockSpec((1, _TILE_N, 3), lambda b, i: (b, i, 0)),
            out_shape=jax.ShapeDtypeStruct((B, H, 3), jnp.int32),
        )(xyz_t_h, qneg2_h, sparse_xyz, sneg)

        flat_idx = jnp.transpose(idx, (2, 0, 1)).reshape(-1)
        g = _sc_gather(table, flat_idx)
        gt = jnp.transpose(g[:, :6].reshape(3, B, H, 6),
                           (1, 0, 3, 2))

        return pl.pallas_call(
            _combine_kernel,
            grid=(B, H // _TILE_W),
            in_specs=[
                pl.BlockSpec((1, 3, 6, _TILE_W), lambda b, i: (b, 0, 0, i)),
                pl.BlockSpec((1, C, _TILE_W), lambda b, i: (b, 0, i)),
            ],
            out_specs=pl.BlockSpec((1, C, _TILE_W), lambda b, i: (b, 0, i)),
            out_shape=jax.ShapeDtypeStruct((B, C, H), jnp.float32),
        )(gt, xyz_h)

    P = 4
    H = N // P
    outs = [run_half(xyz_t[:, i * H:(i + 1) * H],
                     qneg2[:, i * H:(i + 1) * H],
                     xyz[:, :, i * H:(i + 1) * H]) for i in range(P)]
    return jnp.concatenate(outs, axis=2)


# speedup vs baseline: 1.0349x; 1.0349x over previous
import jax
import jax.numpy as jnp
from jax import lax
from jax.experimental import pallas as pl
from jax.experimental.pallas import tpu as pltpu
from jax.experimental.pallas import tpu_sc as plsc

_TILE_N = 1024
_TILE_W = 2048
_GATHER_W = 256


def _select_kernel(xyz_t_ref, qneg2_ref, sxyz_ref, sneg_ref, idx_ref):
    b = pl.program_id(0)
    q = xyz_t_ref[0]
    s = sxyz_ref[0]
    T = q.shape[0]
    S = s.shape[1]

    qc = [q[:, c:c + 1] for c in range(3)]
    sc = [s[c:c + 1, :] for c in range(3)]
    qq = qc[0] * qc[0] + qc[1] * qc[1] + qc[2] * qc[2]
    ss = sc[0] * sc[0] + sc[1] * sc[1] + sc[2] * sc[2]

    crossm = lax.dot_general(
        qneg2_ref[0], sneg_ref[0], (((1,), (0,)), ((), ())),
        preferred_element_type=jnp.float32)
    d_sel = (qq + ss) + crossm

    iota_f = lax.broadcasted_iota(jnp.int32, (T, S), 1).astype(jnp.float32)
    inf = jnp.float32(jnp.inf)
    big = jnp.float32(1e9)
    base_id = b * S

    ids = []
    for k in range(3):
        mk = jnp.min(d_sel, axis=1, keepdims=True)
        cand = jnp.where(d_sel == mk, iota_f, big)
        ik = jnp.min(cand, axis=1, keepdims=True)
        ids.append(ik.astype(jnp.int32) + base_id)
        if k < 2:
            d_sel = jnp.where(cand == ik, inf, d_sel)
    idx_ref[0] = jnp.concatenate(ids, axis=1)


def _combine_kernel(g_ref, xyz_ref, out_ref):
    q = xyz_ref[0]
    recip = []
    flows = []
    for k in range(3):
        arr = g_ref[0, k]
        diff = arr[0:3, :] - q
        sq = diff * diff
        dist2 = sq[0:1, :] + sq[1:2, :] + sq[2:3, :]
        recip.append(1.0 / jnp.maximum(jnp.sqrt(dist2), 1e-10))
        flows.append(arr[3:6, :])
    norm = recip[0] + recip[1] + recip[2]
    acc = (recip[0] / norm) * flows[0]
    acc = acc + (recip[1] / norm) * flows[1]
    acc = acc + (recip[2] / norm) * flows[2]
    out_ref[0] = acc


def _sc_gather(table, indices):
    M = indices.shape[0]
    mesh = plsc.VectorSubcoreMesh(core_axis_name="c", subcore_axis_name="s")
    idx2d = indices.reshape(1, M)

    @pl.kernel(out_type=jax.ShapeDtypeStruct((M, 128), table.dtype), mesh=mesh)
    def gather_kernel(x_hbm, i_hbm, o_hbm):
        def body(i_vmem, o_vmem):
            pltpu.sync_copy(x_hbm.at[i_vmem.at[0]], o_vmem)

        pltpu.emit_pipeline(
            body,
            grid=(M // _GATHER_W,),
            in_specs=[pl.BlockSpec((1, _GATHER_W),
                                   index_map=lambda i: (0, i))],
            out_specs=[pl.BlockSpec((_GATHER_W, 128),
                                    index_map=lambda i: (i, 0))],
            core_axis_name=("c", "s"),
            dimension_semantics=(pltpu.PARALLEL,),
        )(i_hbm, o_hbm)

    return gather_kernel(table, idx2d)


def kernel(xyz, sparse_xyz, sparse_flow):
    B, C, N = xyz.shape
    _, _, S = sparse_xyz.shape
    xyz_t = jnp.transpose(xyz, (0, 2, 1))
    qb = xyz_t.astype(jnp.bfloat16) * jnp.bfloat16(-2.0)
    qneg2 = jnp.pad(qb, ((0, 0), (0, 0), (0, 5)))
    sneg = jnp.pad(sparse_xyz.astype(jnp.bfloat16),
                   ((0, 0), (0, 5), (0, 0)))

    table = jnp.concatenate(
        [jnp.transpose(sparse_xyz, (0, 2, 1)),
         jnp.transpose(sparse_flow, (0, 2, 1))], axis=2)
    table = jnp.pad(table, ((0, 0), (0, 0), (0, 122)))
    table = table.reshape(B * S, 128)

    def run_half(xyz_t_h, qneg2_h, xyz_h):
        H = xyz_t_h.shape[1]
        idx = pl.pallas_call(
            _select_kernel,
            grid=(B, H // _TILE_N),
            in_specs=[
                pl.BlockSpec((1, _TILE_N, C), lambda b, i: (b, i, 0)),
                pl.BlockSpec((1, _TILE_N, 8), lambda b, i: (b, i, 0)),
                pl.BlockSpec((1, C, S), lambda b, i: (b, 0, 0)),
                pl.BlockSpec((1, 8, S), lambda b, i: (b, 0, 0)),
            ],
            out_specs=pl.BlockSpec((1, _TILE_N, 3), lambda b, i: (b, i, 0)),
            out_shape=jax.ShapeDtypeStruct((B, H, 3), jnp.int32),
        )(xyz_t_h, qneg2_h, sparse_xyz, sneg)

        flat_idx = jnp.transpose(idx, (2, 0, 1)).reshape(-1)
        g = _sc_gather(table, flat_idx)
        gt = jnp.transpose(g[:, :6].reshape(3, B, H, 6),
                           (1, 0, 3, 2))

        return pl.pallas_call(
            _combine_kernel,
            grid=(B, H // _TILE_W),
            in_specs=[
                pl.BlockSpec((1, 3, 6, _TILE_W), lambda b, i: (b, 0, 0, i)),
                pl.BlockSpec((1, C, _TILE_W), lambda b, i: (b, 0, i)),
            ],
            out_specs=pl.BlockSpec((1, C, _TILE_W), lambda b, i: (b, 0, i)),
            out_shape=jax.ShapeDtypeStruct((B, C, H), jnp.float32),
        )(gt, xyz_h)

    P = 2
    H = N // P
    outs = [run_half(xyz_t[:, i * H:(i + 1) * H],
                     qneg2[:, i * H:(i + 1) * H],
                     xyz[:, :, i * H:(i + 1) * H]) for i in range(P)]
    return jnp.concatenate(outs, axis=2)
